# baseline (speedup 1.0000x reference)
import functools

import jax
import jax.numpy as jnp
from jax import lax
from jax.experimental import pallas as pl
from jax.experimental.pallas import tpu as pltpu
from jax.experimental.pallas import tpu_sc as plsc

_N_FIELDS = 26
_VOCAB = 100000
_DIM = 64
_BATCH = 16384
_NC = 2
_NS = 16
_NW = _NC * _NS
_BPW = _BATCH // _NW
_LANES = 16
_FH = _N_FIELDS // 2


def _make_body(nf):
    def _sc_body(x_hbm, tab_hbm, out_hbm, *rest):
        idx_v = rest[0]
        idx_f = rest[1:1 + nf]
        acc = rest[1 + nf]
        sem = rest[2 + nf]

        c = lax.axis_index("c")
        s = lax.axis_index("s")
        wid = s * _NC + c
        base = wid * _BPW

        for f in range(nf):
            pltpu.sync_copy(x_hbm.at[pl.ds(f * _BATCH + base, _BPW)],
                            idx_v.at[pl.ds(f * _BPW, _BPW)])

        zeros = jnp.zeros((_LANES,), jnp.float32)

        def zero_body(i, _):
            for j in range(_DIM // _LANES):
                acc[i, pl.ds(j * _LANES, _LANES)] = zeros
            return 0
        lax.fori_loop(0, _BPW, zero_body, 0)

        copies = []
        for f in range(nf):
            dst = idx_f[f]

            def fill_body(i, _, f=f, dst=dst):
                sl = pl.ds(i * _LANES, _LANES)
                sl_src = pl.ds(f * _BPW + i * _LANES, _LANES)
                dst[sl] = idx_v[sl_src] + f * _VOCAB
                return 0
            lax.fori_loop(0, _BPW // _LANES, fill_body, 0)
            copies.append(pltpu.async_copy(tab_hbm.at[dst], acc, sem,
                                           add=True))

        for cp in copies:
            cp.wait()

        pltpu.sync_copy(acc, out_hbm.at[pl.ds(base, _BPW)])

    return _sc_body


def _half_call(nf):
    scratch = [pltpu.VMEM((nf * _BPW,), jnp.int32)]
    scratch += [pltpu.VMEM((_BPW,), jnp.int32) for _ in range(nf)]
    scratch += [pltpu.VMEM((_BPW, _DIM), jnp.float32), pltpu.SemaphoreType.DMA]
    return functools.partial(
        pl.kernel,
        out_type=jax.ShapeDtypeStruct((_BATCH, _DIM), jnp.float32),
        mesh=plsc.VectorSubcoreMesh(core_axis_name="c", subcore_axis_name="s"),
        compiler_params=pltpu.CompilerParams(use_tc_tiling_on_sc=False),
        scratch_types=scratch,
    )(_make_body(nf))


@jax.jit
def kernel(x, tables):
    xt = x.T
    xa = xt[:_FH].reshape(_FH * _BATCH)
    xb = xt[_FH:].reshape(_FH * _BATCH)
    taba = tables[:_FH].reshape(_FH * _VOCAB, _DIM)
    tabb = tables[_FH:].reshape(_FH * _VOCAB, _DIM)
    outa = _half_call(_FH)(xa, taba)
    outb = _half_call(_FH)(xb, tabb)
    return outa + outb


# speedup vs baseline: 1.5380x; 1.5380x over previous
import functools

import jax
import jax.numpy as jnp
from jax import lax
from jax.experimental import pallas as pl
from jax.experimental.pallas import tpu as pltpu
from jax.experimental.pallas import tpu_sc as plsc

_N_FIELDS = 26
_VOCAB = 100000
_DIM = 64
_BATCH = 16384
_NC = 2
_NS = 16
_NW = _NC * _NS
_BPW = _BATCH // _NW
_LANES = 16


def _sc_body(x_hbm, tab_hbm, out_hbm, *rest):
    idx_v = rest[0]
    idx_f = rest[1:1 + _N_FIELDS]
    acc = rest[1 + _N_FIELDS]
    sem = rest[2 + _N_FIELDS]

    c = lax.axis_index("c")
    s = lax.axis_index("s")
    wid = s * _NC + c
    base = wid * _BPW

    for f in range(_N_FIELDS):
        pltpu.sync_copy(x_hbm.at[pl.ds(f * _BATCH + base, _BPW)],
                        idx_v.at[pl.ds(f * _BPW, _BPW)])

    zeros = jnp.zeros((_LANES,), jnp.float32)

    def zero_body(i, _):
        for j in range(_DIM // _LANES):
            acc[i, pl.ds(j * _LANES, _LANES)] = zeros
        return 0
    lax.fori_loop(0, _BPW, zero_body, 0)

    copies = []
    for f in range(_N_FIELDS):
        dst = idx_f[f]

        def fill_body(i, _, f=f, dst=dst):
            sl = pl.ds(i * _LANES, _LANES)
            sl_src = pl.ds(f * _BPW + i * _LANES, _LANES)
            dst[sl] = idx_v[sl_src] + f * _VOCAB
            return 0
        lax.fori_loop(0, _BPW // _LANES, fill_body, 0)
        copies.append(pltpu.async_copy(tab_hbm.at[dst], acc, sem, add=True))

    for cp in copies:
        cp.wait()

    pltpu.sync_copy(acc, out_hbm.at[pl.ds(base, _BPW)])


@jax.jit
def kernel(x, tables):
    x1 = x.T.reshape(_N_FIELDS * _BATCH)
    tab = tables.reshape(_N_FIELDS * _VOCAB, _DIM)
    scratch = [pltpu.VMEM((_N_FIELDS * _BPW,), jnp.int32)]
    scratch += [pltpu.VMEM((_BPW,), jnp.int32) for _ in range(_N_FIELDS)]
    scratch += [pltpu.VMEM((_BPW, _DIM), jnp.float32), pltpu.SemaphoreType.DMA]
    run = functools.partial(
        pl.kernel,
        out_type=jax.ShapeDtypeStruct((_BATCH, _DIM), jnp.float32),
        mesh=plsc.VectorSubcoreMesh(core_axis_name="c", subcore_axis_name="s"),
        compiler_params=pltpu.CompilerParams(use_tc_tiling_on_sc=False),
        scratch_types=scratch,
    )(_sc_body)
    return run(x1, tab)
